# baseline (speedup 1.0000x reference)
import functools

import jax
import jax.numpy as jnp
from jax import lax
from jax.experimental import pallas as pl
from jax.experimental.pallas import tpu as pltpu
from jax.experimental.pallas import tpu_sc as plsc

S = 4096
D = 1024
E = 8
DFF = 4096
CAP = 640
RPE = 656
NROWS = E * RPE
SENT = 648
TB = 512
LW = 128
NJ = 4
DFB = DFF // NJ

NW = 32
CH = 32
NCH = (S // NW) // CH
NBUF = 3

ROUTER_WEIGHT = 0.001
EXPERTS_WEIGHT = 0.01


def _router_body(h_ref, gw_ref, dst_ref, loss_ref, acc_ref):
    i = pl.program_id(0)
    nb = pl.num_programs(0)

    @pl.when(i == 0)
    def _():
        acc_ref[...] = jnp.zeros_like(acc_ref)

    h = h_ref[...]
    logits = jnp.dot(h, gw_ref[...], preferred_element_type=jnp.float32)
    lane = lax.broadcasted_iota(jnp.int32, (TB, LW), 1)
    neg_inf = jnp.float32(-jnp.inf)
    logits = jnp.where(lane < E, logits, neg_inf)

    m1 = jnp.max(logits, axis=1, keepdims=True)
    i1 = jnp.min(jnp.where(logits == m1, lane, LW), axis=1, keepdims=True)
    l2 = jnp.where(lane == i1, neg_inf, logits)
    m2 = jnp.max(l2, axis=1, keepdims=True)
    i2 = jnp.min(jnp.where(l2 == m2, lane, LW), axis=1, keepdims=True)

    ex = jnp.exp(logits - m1)
    se = jnp.sum(ex, axis=1, keepdims=True)
    probs = ex / se
    lse = m1 + jnp.log(se)

    oh1 = (lane == i1).astype(jnp.float32)
    oh2 = (lane == i2).astype(jnp.float32)
    d_oh = oh1 + oh2

    r_i = lax.broadcasted_iota(jnp.int32, (TB, TB), 0)
    c_i = lax.broadcasted_iota(jnp.int32, (TB, TB), 1)
    tri = (r_i >= c_i).astype(jnp.float32)

    pos = jnp.dot(tri, d_oh, preferred_element_type=jnp.float32) + acc_ref[0:1, :]
    pos1 = jnp.sum(pos * oh1, axis=1, keepdims=True)
    pos2 = jnp.sum(pos * oh2, axis=1, keepdims=True)
    cand1 = jnp.where(pos1 <= CAP, i1, -1)
    cand2 = jnp.where(pos2 <= CAP, i2, -1)
    assigned = jnp.maximum(cand1, cand2)

    a_oh = (lane == assigned).astype(jnp.float32)
    spos = jnp.dot(tri, a_oh, preferred_element_type=jnp.float32) + acc_ref[1:2, :]
    slot = jnp.sum(spos * a_oh, axis=1, keepdims=True).astype(jnp.int32) - 1
    dst_ref[...] = jnp.where(assigned >= 0, assigned * RPE + slot, SENT)

    acc_ref[0:1, :] = acc_ref[0:1, :] + jnp.sum(d_oh, axis=0, keepdims=True)
    acc_ref[1:2, :] = acc_ref[1:2, :] + jnp.sum(a_oh, axis=0, keepdims=True)
    acc_ref[2:3, :] = acc_ref[2:3, :] + jnp.sum(probs, axis=0, keepdims=True)
    acc_ref[3:4, :] = acc_ref[3:4, :] + jnp.sum(lse * lse)

    @pl.when(i == nb - 1)
    def _():
        sf = jnp.float32(S)
        load = acc_ref[0:1, :] / sf
        imp = acc_ref[2:3, :] / sf
        bal = EXPERTS_WEIGHT * jnp.sum(load * imp)
        rl = ROUTER_WEIGHT * jnp.max(acc_ref[3:4, :]) / sf
        loss_ref[...] = jnp.reshape(rl + bal, (1, 1))


def _ffn_body(x_ref, w1_ref, b1_ref, w2_ref, b2_ref, y_ref):
    j = pl.program_id(1)
    x = x_ref[0].astype(jnp.bfloat16)
    h1 = lax.dot_general(x, w1_ref[0], (((1,), (1,)), ((), ())),
                         preferred_element_type=jnp.float32)
    h1 = h1 + b1_ref[0]
    h1 = 0.5 * h1 * (1.0 + lax.erf(h1 * jnp.float32(0.7071067811865476)))
    contrib = lax.dot_general(h1.astype(jnp.bfloat16), w2_ref[0],
                              (((1,), (1,)), ((), ())),
                              preferred_element_type=jnp.float32)

    @pl.when(j == 0)
    def _():
        y_ref[0] = contrib + b2_ref[0]

    @pl.when(j > 0)
    def _():
        y_ref[0] = y_ref[0] + contrib

    @pl.when(j == NJ - 1)
    def _():
        row = lax.broadcasted_iota(jnp.int32, (RPE, D), 0)
        y_ref[0] = jnp.where(row < CAP, y_ref[0], 0.0)


def _sc_wid():
    return lax.axis_index("s") * 2 + lax.axis_index("c")


def _dispatch_body(h_hbm, dst_hbm, x_hbm, idx_v, b0, b1, b2, s0, s1, s2,
                   t0, t1, t2):
    wid = _sc_wid()
    base = wid * (NCH * CH)
    bufs, csems, ssems = (b0, b1, b2), (s0, s1, s2), (t0, t1, t2)
    cps, scs = [None] * NCH, [None] * NCH
    for c in range(NBUF):
        cps[c] = pltpu.async_copy(
            h_hbm.at[pl.ds(base + c * CH, CH)], bufs[c], csems[c])
    pltpu.sync_copy(dst_hbm.at[wid], idx_v)
    for c in range(NCH):
        cps[c].wait()
        scs[c] = pltpu.async_copy(
            bufs[c % NBUF], x_hbm.at[idx_v.at[c]], ssems[c % NBUF])
        n = c + NBUF
        if n < NCH:
            scs[c].wait()
            cps[n] = pltpu.async_copy(
                h_hbm.at[pl.ds(base + n * CH, CH)], bufs[n % NBUF],
                csems[n % NBUF])
    for c in range(max(0, NCH - NBUF), NCH):
        scs[c].wait()


def _combine_body(y_hbm, dst_hbm, o_hbm, idx_v, b0, b1, b2, s0, s1, s2,
                  t0, t1, t2):
    wid = _sc_wid()
    base = wid * (NCH * CH)
    bufs, gsems, wsems = (b0, b1, b2), (s0, s1, s2), (t0, t1, t2)
    pltpu.sync_copy(dst_hbm.at[wid], idx_v)
    gts, wrs = [None] * NCH, [None] * NCH
    for c in range(NBUF):
        gts[c] = pltpu.async_copy(
            y_hbm.at[idx_v.at[c]], bufs[c], gsems[c])
    for c in range(NCH):
        gts[c].wait()
        wrs[c] = pltpu.async_copy(
            bufs[c % NBUF], o_hbm.at[pl.ds(base + c * CH, CH)],
            wsems[c % NBUF])
        n = c + NBUF
        if n < NCH:
            wrs[c].wait()
            gts[n] = pltpu.async_copy(
                y_hbm.at[idx_v.at[n]], bufs[n % NBUF], gsems[n % NBUF])
    for c in range(max(0, NCH - NBUF), NCH):
        wrs[c].wait()


def _make_router():
    return pl.pallas_call(
        _router_body,
        grid=(S // TB,),
        in_specs=[
            pl.BlockSpec((TB, D), lambda i: (i, 0)),
            pl.BlockSpec((D, LW), lambda i: (0, 0)),
        ],
        out_specs=[
            pl.BlockSpec((TB, 1), lambda i: (i, 0)),
            pl.BlockSpec((1, 1), lambda i: (0, 0)),
        ],
        out_shape=[
            jax.ShapeDtypeStruct((S, 1), jnp.int32),
            jax.ShapeDtypeStruct((1, 1), jnp.float32),
        ],
        scratch_shapes=[pltpu.VMEM((8, LW), jnp.float32)],
        compiler_params=pltpu.CompilerParams(
            dimension_semantics=("arbitrary",)),
    )


def _make_ffn():
    return pl.pallas_call(
        _ffn_body,
        grid=(E, NJ),
        in_specs=[
            pl.BlockSpec((1, RPE, D), lambda e, j: (e, 0, 0)),
            pl.BlockSpec((1, DFB, D), lambda e, j: (e, j, 0)),
            pl.BlockSpec((1, 1, DFB), lambda e, j: (e, 0, j)),
            pl.BlockSpec((1, D, DFB), lambda e, j: (e, 0, j)),
            pl.BlockSpec((1, 1, D), lambda e, j: (e, 0, 0)),
        ],
        out_specs=pl.BlockSpec((1, RPE, D), lambda e, j: (e, 0, 0)),
        out_shape=jax.ShapeDtypeStruct((E, RPE, D), jnp.float32),
        compiler_params=pltpu.CompilerParams(
            dimension_semantics=("arbitrary", "arbitrary")),
    )


def _sc_mesh():
    return plsc.VectorSubcoreMesh(core_axis_name="c", subcore_axis_name="s")


def _make_dispatch():
    return functools.partial(
        pl.kernel,
        out_type=jax.ShapeDtypeStruct((NROWS, D), jnp.float32),
        mesh=_sc_mesh(),
        scratch_types=[
            pltpu.VMEM((NCH, CH), jnp.int32),
            pltpu.VMEM((CH, D), jnp.float32),
            pltpu.VMEM((CH, D), jnp.float32),
            pltpu.VMEM((CH, D), jnp.float32),
            pltpu.SemaphoreType.DMA,
            pltpu.SemaphoreType.DMA,
            pltpu.SemaphoreType.DMA,
            pltpu.SemaphoreType.DMA,
            pltpu.SemaphoreType.DMA,
            pltpu.SemaphoreType.DMA,
        ],
    )(_dispatch_body)


def _make_combine():
    return functools.partial(
        pl.kernel,
        out_type=jax.ShapeDtypeStruct((S, D), jnp.float32),
        mesh=_sc_mesh(),
        scratch_types=[
            pltpu.VMEM((NCH, CH), jnp.int32),
            pltpu.VMEM((CH, D), jnp.float32),
            pltpu.VMEM((CH, D), jnp.float32),
            pltpu.VMEM((CH, D), jnp.float32),
            pltpu.SemaphoreType.DMA,
            pltpu.SemaphoreType.DMA,
            pltpu.SemaphoreType.DMA,
            pltpu.SemaphoreType.DMA,
            pltpu.SemaphoreType.DMA,
            pltpu.SemaphoreType.DMA,
        ],
    )(_combine_body)


def kernel(H, gate_W, fc1_w, fc1_b, fc2_w, fc2_b):
    b, t, d = H.shape
    h2 = H.reshape(S, D)
    gwt = jnp.zeros((D, LW), jnp.float32).at[:, :E].set(gate_W.T)

    dst, loss = _make_router()(h2, gwt)
    dst3 = dst.reshape(NW, NCH, CH)

    x = _make_dispatch()(h2, dst3)
    y = _make_ffn()(x.reshape(E, RPE, D), fc1_w.astype(jnp.bfloat16),
                    fc1_b.reshape(E, 1, DFF),
                    fc2_w.astype(jnp.bfloat16), fc2_b.reshape(E, 1, D))
    out = _make_combine()(y.reshape(NROWS, D), dst3)
    return out.reshape(b, t, d), loss[0, 0]


# speedup vs baseline: 1.3307x; 1.3307x over previous
import functools

import jax
import jax.numpy as jnp
from jax import lax
from jax.experimental import pallas as pl
from jax.experimental.pallas import tpu as pltpu
from jax.experimental.pallas import tpu_sc as plsc

S = 4096
D = 1024
E = 8
DFF = 4096
CAP = 640
RPE = 656
NROWS = E * RPE
SENT = 648
TB = 512
LW = 128
NJ = 4
DFB = DFF // NJ

NW = 32
CH = 32
NCH = (S // NW) // CH
NBUF = 3

ROUTER_WEIGHT = 0.001
EXPERTS_WEIGHT = 0.01


def _router_body(h_ref, gw_ref, dst_ref, loss_ref, acc_ref):
    i = pl.program_id(0)
    nb = pl.num_programs(0)

    @pl.when(i == 0)
    def _():
        acc_ref[...] = jnp.zeros_like(acc_ref)

    h = h_ref[...]
    logits = jnp.dot(h, gw_ref[...], preferred_element_type=jnp.float32)
    lane = lax.broadcasted_iota(jnp.int32, (TB, LW), 1)
    neg_inf = jnp.float32(-jnp.inf)
    logits = jnp.where(lane < E, logits, neg_inf)

    m1 = jnp.max(logits, axis=1, keepdims=True)
    i1 = jnp.min(jnp.where(logits == m1, lane, LW), axis=1, keepdims=True)
    l2 = jnp.where(lane == i1, neg_inf, logits)
    m2 = jnp.max(l2, axis=1, keepdims=True)
    i2 = jnp.min(jnp.where(l2 == m2, lane, LW), axis=1, keepdims=True)

    ex = jnp.exp(logits - m1)
    se = jnp.sum(ex, axis=1, keepdims=True)
    probs = ex / se
    lse = m1 + jnp.log(se)

    oh1 = (lane == i1).astype(jnp.float32)
    oh2 = (lane == i2).astype(jnp.float32)
    d_oh = oh1 + oh2

    r_i = lax.broadcasted_iota(jnp.int32, (TB, TB), 0)
    c_i = lax.broadcasted_iota(jnp.int32, (TB, TB), 1)
    tri = (r_i >= c_i).astype(jnp.float32)

    pos = jnp.dot(tri, d_oh, preferred_element_type=jnp.float32) + acc_ref[0:1, :]
    pos1 = jnp.sum(pos * oh1, axis=1, keepdims=True)
    pos2 = jnp.sum(pos * oh2, axis=1, keepdims=True)
    cand1 = jnp.where(pos1 <= CAP, i1, -1)
    cand2 = jnp.where(pos2 <= CAP, i2, -1)
    assigned = jnp.maximum(cand1, cand2)

    a_oh = (lane == assigned).astype(jnp.float32)
    spos = jnp.dot(tri, a_oh, preferred_element_type=jnp.float32) + acc_ref[1:2, :]
    slot = jnp.sum(spos * a_oh, axis=1, keepdims=True).astype(jnp.int32) - 1
    dst_ref[...] = jnp.where(assigned >= 0, assigned * RPE + slot, SENT)

    acc_ref[0:1, :] = acc_ref[0:1, :] + jnp.sum(d_oh, axis=0, keepdims=True)
    acc_ref[1:2, :] = acc_ref[1:2, :] + jnp.sum(a_oh, axis=0, keepdims=True)
    acc_ref[2:3, :] = acc_ref[2:3, :] + jnp.sum(probs, axis=0, keepdims=True)
    acc_ref[3:4, :] = acc_ref[3:4, :] + jnp.sum(lse * lse)

    @pl.when(i == nb - 1)
    def _():
        sf = jnp.float32(S)
        load = acc_ref[0:1, :] / sf
        imp = acc_ref[2:3, :] / sf
        bal = EXPERTS_WEIGHT * jnp.sum(load * imp)
        rl = ROUTER_WEIGHT * jnp.max(acc_ref[3:4, :]) / sf
        loss_ref[...] = jnp.reshape(rl + bal, (1, 1))


def _ffn_body(x_ref, w1_ref, b1_ref, w2_ref, b2_ref, y_ref):
    j = pl.program_id(1)
    x = x_ref[0]
    h1 = lax.dot_general(x, w1_ref[0], (((1,), (1,)), ((), ())),
                         preferred_element_type=jnp.float32)
    h1 = h1 + b1_ref[0]
    h1 = 0.5 * h1 * (1.0 + lax.erf(h1 * jnp.float32(0.7071067811865476)))
    contrib = lax.dot_general(h1, w2_ref[0], (((1,), (1,)), ((), ())),
                              preferred_element_type=jnp.float32)

    @pl.when(j == 0)
    def _():
        y_ref[0] = contrib + b2_ref[0]

    @pl.when(j > 0)
    def _():
        y_ref[0] = y_ref[0] + contrib

    @pl.when(j == NJ - 1)
    def _():
        row = lax.broadcasted_iota(jnp.int32, (RPE, D), 0)
        y_ref[0] = jnp.where(row < CAP, y_ref[0], 0.0)


def _sc_wid():
    return lax.axis_index("s") * 2 + lax.axis_index("c")


def _dispatch_body(h_hbm, dst_hbm, x_hbm, idx_v, b0, b1, b2, s0, s1, s2,
                   t0, t1, t2):
    wid = _sc_wid()
    base = wid * (NCH * CH)
    bufs, csems, ssems = (b0, b1, b2), (s0, s1, s2), (t0, t1, t2)
    cps, scs = [None] * NCH, [None] * NCH
    for c in range(NBUF):
        cps[c] = pltpu.async_copy(
            h_hbm.at[pl.ds(base + c * CH, CH)], bufs[c], csems[c])
    pltpu.sync_copy(dst_hbm.at[wid], idx_v)
    for c in range(NCH):
        cps[c].wait()
        scs[c] = pltpu.async_copy(
            bufs[c % NBUF], x_hbm.at[idx_v.at[c]], ssems[c % NBUF])
        n = c + NBUF
        if n < NCH:
            scs[c].wait()
            cps[n] = pltpu.async_copy(
                h_hbm.at[pl.ds(base + n * CH, CH)], bufs[n % NBUF],
                csems[n % NBUF])
    for c in range(max(0, NCH - NBUF), NCH):
        scs[c].wait()


def _combine_body(y_hbm, dst_hbm, o_hbm, idx_v, b0, b1, b2, s0, s1, s2,
                  t0, t1, t2):
    wid = _sc_wid()
    base = wid * (NCH * CH)
    bufs, gsems, wsems = (b0, b1, b2), (s0, s1, s2), (t0, t1, t2)
    pltpu.sync_copy(dst_hbm.at[wid], idx_v)
    gts, wrs = [None] * NCH, [None] * NCH
    for c in range(NBUF):
        gts[c] = pltpu.async_copy(
            y_hbm.at[idx_v.at[c]], bufs[c], gsems[c])
    for c in range(NCH):
        gts[c].wait()
        wrs[c] = pltpu.async_copy(
            bufs[c % NBUF], o_hbm.at[pl.ds(base + c * CH, CH)],
            wsems[c % NBUF])
        n = c + NBUF
        if n < NCH:
            wrs[c].wait()
            gts[n] = pltpu.async_copy(
                y_hbm.at[idx_v.at[n]], bufs[n % NBUF], gsems[n % NBUF])
    for c in range(max(0, NCH - NBUF), NCH):
        wrs[c].wait()


def _make_router():
    return pl.pallas_call(
        _router_body,
        grid=(S // TB,),
        in_specs=[
            pl.BlockSpec((TB, D), lambda i: (i, 0)),
            pl.BlockSpec((D, LW), lambda i: (0, 0)),
        ],
        out_specs=[
            pl.BlockSpec((TB, 1), lambda i: (i, 0)),
            pl.BlockSpec((1, 1), lambda i: (0, 0)),
        ],
        out_shape=[
            jax.ShapeDtypeStruct((S, 1), jnp.int32),
            jax.ShapeDtypeStruct((1, 1), jnp.float32),
        ],
        scratch_shapes=[pltpu.VMEM((8, LW), jnp.float32)],
        compiler_params=pltpu.CompilerParams(
            dimension_semantics=("arbitrary",)),
    )


def _make_ffn():
    return pl.pallas_call(
        _ffn_body,
        grid=(E, NJ),
        in_specs=[
            pl.BlockSpec((1, RPE, D), lambda e, j: (e, 0, 0)),
            pl.BlockSpec((1, DFB, D), lambda e, j: (e, j, 0)),
            pl.BlockSpec((1, 1, DFB), lambda e, j: (e, 0, j)),
            pl.BlockSpec((1, D, DFB), lambda e, j: (e, 0, j)),
            pl.BlockSpec((1, 1, D), lambda e, j: (e, 0, 0)),
        ],
        out_specs=pl.BlockSpec((1, RPE, D), lambda e, j: (e, 0, 0)),
        out_shape=jax.ShapeDtypeStruct((E, RPE, D), jnp.float32),
        compiler_params=pltpu.CompilerParams(
            dimension_semantics=("arbitrary", "arbitrary")),
    )


def _sc_mesh():
    return plsc.VectorSubcoreMesh(core_axis_name="c", subcore_axis_name="s")


def _make_dispatch():
    return functools.partial(
        pl.kernel,
        out_type=jax.ShapeDtypeStruct((NROWS, D), jnp.float32),
        mesh=_sc_mesh(),
        scratch_types=[
            pltpu.VMEM((NCH, CH), jnp.int32),
            pltpu.VMEM((CH, D), jnp.float32),
            pltpu.VMEM((CH, D), jnp.float32),
            pltpu.VMEM((CH, D), jnp.float32),
            pltpu.SemaphoreType.DMA,
            pltpu.SemaphoreType.DMA,
            pltpu.SemaphoreType.DMA,
            pltpu.SemaphoreType.DMA,
            pltpu.SemaphoreType.DMA,
            pltpu.SemaphoreType.DMA,
        ],
    )(_dispatch_body)


def _make_combine():
    return functools.partial(
        pl.kernel,
        out_type=jax.ShapeDtypeStruct((S, D), jnp.float32),
        mesh=_sc_mesh(),
        scratch_types=[
            pltpu.VMEM((NCH, CH), jnp.int32),
            pltpu.VMEM((CH, D), jnp.float32),
            pltpu.VMEM((CH, D), jnp.float32),
            pltpu.VMEM((CH, D), jnp.float32),
            pltpu.SemaphoreType.DMA,
            pltpu.SemaphoreType.DMA,
            pltpu.SemaphoreType.DMA,
            pltpu.SemaphoreType.DMA,
            pltpu.SemaphoreType.DMA,
            pltpu.SemaphoreType.DMA,
        ],
    )(_combine_body)


def kernel(H, gate_W, fc1_w, fc1_b, fc2_w, fc2_b):
    b, t, d = H.shape
    h2 = H.reshape(S, D)
    gwt = jnp.zeros((D, LW), jnp.float32).at[:, :E].set(gate_W.T)

    dst, loss = _make_router()(h2, gwt)
    dst3 = dst.reshape(NW, NCH, CH)

    x = _make_dispatch()(h2, dst3)
    y = _make_ffn()(x.reshape(E, RPE, D), fc1_w,
                    fc1_b.reshape(E, 1, DFF), fc2_w, fc2_b.reshape(E, 1, D))
    out = _make_combine()(y.reshape(NROWS, D), dst3)
    return out.reshape(b, t, d), loss[0, 0]


# speedup vs baseline: 1.3491x; 1.0138x over previous
import functools

import jax
import jax.numpy as jnp
from jax import lax
from jax.experimental import pallas as pl
from jax.experimental.pallas import tpu as pltpu
from jax.experimental.pallas import tpu_sc as plsc

S = 4096
D = 1024
E = 8
DFF = 4096
CAP = 640
RPE = 656
NROWS = E * RPE
SENT = 648
TB = 512
LW = 128
NJ = 4
DFB = DFF // NJ
DP = D // 2

NW = 32
CH = 32
NCH = (S // NW) // CH
NBUF = 3

ROUTER_WEIGHT = 0.001
EXPERTS_WEIGHT = 0.01


def _pack_pair(lo_bf, hi_bf):
    lo = pltpu.bitcast(lo_bf, jnp.uint16).astype(jnp.uint32)
    hi = pltpu.bitcast(hi_bf, jnp.uint16).astype(jnp.uint32)
    return pltpu.bitcast(lo | (hi << 16), jnp.float32)


def _unpack_pair(x_f32):
    w = pltpu.bitcast(x_f32, jnp.uint32)
    lo = pltpu.bitcast((w & 0xFFFF).astype(jnp.uint16), jnp.bfloat16)
    hi = pltpu.bitcast((w >> 16).astype(jnp.uint16), jnp.bfloat16)
    return lo, hi


def _router_body(h_ref, gw_ref, dst_ref, loss_ref, hbf_ref, acc_ref):
    i = pl.program_id(0)
    nb = pl.num_programs(0)

    @pl.when(i == 0)
    def _():
        acc_ref[...] = jnp.zeros_like(acc_ref)

    h = h_ref[...]
    hb = h.astype(jnp.bfloat16)
    hbf_ref[...] = _pack_pair(hb[:, :DP], hb[:, DP:])
    logits = jnp.dot(h, gw_ref[...], preferred_element_type=jnp.float32)
    lane = lax.broadcasted_iota(jnp.int32, (TB, LW), 1)
    neg_inf = jnp.float32(-jnp.inf)
    logits = jnp.where(lane < E, logits, neg_inf)

    m1 = jnp.max(logits, axis=1, keepdims=True)
    i1 = jnp.min(jnp.where(logits == m1, lane, LW), axis=1, keepdims=True)
    l2 = jnp.where(lane == i1, neg_inf, logits)
    m2 = jnp.max(l2, axis=1, keepdims=True)
    i2 = jnp.min(jnp.where(l2 == m2, lane, LW), axis=1, keepdims=True)

    ex = jnp.exp(logits - m1)
    se = jnp.sum(ex, axis=1, keepdims=True)
    probs = ex / se
    lse = m1 + jnp.log(se)

    oh1 = (lane == i1).astype(jnp.float32)
    oh2 = (lane == i2).astype(jnp.float32)
    d_oh = oh1 + oh2

    r_i = lax.broadcasted_iota(jnp.int32, (TB, TB), 0)
    c_i = lax.broadcasted_iota(jnp.int32, (TB, TB), 1)
    tri = (r_i >= c_i).astype(jnp.float32)

    pos = jnp.dot(tri, d_oh, preferred_element_type=jnp.float32) + acc_ref[0:1, :]
    pos1 = jnp.sum(pos * oh1, axis=1, keepdims=True)
    pos2 = jnp.sum(pos * oh2, axis=1, keepdims=True)
    cand1 = jnp.where(pos1 <= CAP, i1, -1)
    cand2 = jnp.where(pos2 <= CAP, i2, -1)
    assigned = jnp.maximum(cand1, cand2)

    a_oh = (lane == assigned).astype(jnp.float32)
    spos = jnp.dot(tri, a_oh, preferred_element_type=jnp.float32) + acc_ref[1:2, :]
    slot = jnp.sum(spos * a_oh, axis=1, keepdims=True).astype(jnp.int32) - 1
    dst_ref[...] = jnp.where(assigned >= 0, assigned * RPE + slot, SENT)

    acc_ref[0:1, :] = acc_ref[0:1, :] + jnp.sum(d_oh, axis=0, keepdims=True)
    acc_ref[1:2, :] = acc_ref[1:2, :] + jnp.sum(a_oh, axis=0, keepdims=True)
    acc_ref[2:3, :] = acc_ref[2:3, :] + jnp.sum(probs, axis=0, keepdims=True)
    acc_ref[3:4, :] = acc_ref[3:4, :] + jnp.sum(lse * lse)

    @pl.when(i == nb - 1)
    def _():
        sf = jnp.float32(S)
        load = acc_ref[0:1, :] / sf
        imp = acc_ref[2:3, :] / sf
        bal = EXPERTS_WEIGHT * jnp.sum(load * imp)
        rl = ROUTER_WEIGHT * jnp.max(acc_ref[3:4, :]) / sf
        loss_ref[...] = jnp.reshape(rl + bal, (1, 1))


def _ffn_body(x_ref, w1_ref, b1_ref, w2_ref, b2_ref, y_ref, acc_ref):
    j = pl.program_id(1)
    x = jnp.concatenate(_unpack_pair(x_ref[0]), axis=1)
    h1 = lax.dot_general(x, w1_ref[0].astype(jnp.bfloat16),
                         (((1,), (1,)), ((), ())),
                         preferred_element_type=jnp.float32)
    h1 = h1 + b1_ref[0]
    h1 = 0.5 * h1 * (1.0 + lax.erf(h1 * jnp.float32(0.7071067811865476)))
    contrib = lax.dot_general(h1.astype(jnp.bfloat16),
                              w2_ref[0].astype(jnp.bfloat16),
                              (((1,), (1,)), ((), ())),
                              preferred_element_type=jnp.float32)

    @pl.when(j == 0)
    def _():
        acc_ref[...] = contrib + b2_ref[0]

    @pl.when(j > 0)
    def _():
        acc_ref[...] = acc_ref[...] + contrib

    @pl.when(j == NJ - 1)
    def _():
        row = lax.broadcasted_iota(jnp.int32, (RPE, D), 0)
        yv = jnp.where(row < CAP, acc_ref[...], 0.0).astype(jnp.bfloat16)
        y_ref[0] = _pack_pair(yv[:, :DP], yv[:, DP:])


def _sc_wid():
    return lax.axis_index("s") * 2 + lax.axis_index("c")


def _dispatch_body(h_hbm, dst_hbm, x_hbm, idx_v, b0, b1, b2, s0, s1, s2,
                   t0, t1, t2):
    wid = _sc_wid()
    base = wid * (NCH * CH)
    bufs, csems, ssems = (b0, b1, b2), (s0, s1, s2), (t0, t1, t2)
    cps, scs = [None] * NCH, [None] * NCH
    for c in range(NBUF):
        cps[c] = pltpu.async_copy(
            h_hbm.at[pl.ds(base + c * CH, CH)], bufs[c], csems[c])
    pltpu.sync_copy(dst_hbm.at[wid], idx_v)
    for c in range(NCH):
        cps[c].wait()
        scs[c] = pltpu.async_copy(
            bufs[c % NBUF], x_hbm.at[idx_v.at[c]], ssems[c % NBUF])
        n = c + NBUF
        if n < NCH:
            scs[c].wait()
            cps[n] = pltpu.async_copy(
                h_hbm.at[pl.ds(base + n * CH, CH)], bufs[n % NBUF],
                csems[n % NBUF])
    for c in range(max(0, NCH - NBUF), NCH):
        scs[c].wait()


def _combine_body(y_hbm, dst_hbm, o_hbm, idx_v, b0, b1, b2, s0, s1, s2,
                  t0, t1, t2):
    wid = _sc_wid()
    base = wid * (NCH * CH)
    bufs, gsems, wsems = (b0, b1, b2), (s0, s1, s2), (t0, t1, t2)
    pltpu.sync_copy(dst_hbm.at[wid], idx_v)
    gts, wrs = [None] * NCH, [None] * NCH
    for c in range(NBUF):
        gts[c] = pltpu.async_copy(
            y_hbm.at[idx_v.at[c]], bufs[c], gsems[c])
    for c in range(NCH):
        gts[c].wait()
        wrs[c] = pltpu.async_copy(
            bufs[c % NBUF], o_hbm.at[pl.ds(base + c * CH, CH)],
            wsems[c % NBUF])
        n = c + NBUF
        if n < NCH:
            wrs[c].wait()
            gts[n] = pltpu.async_copy(
                y_hbm.at[idx_v.at[n]], bufs[n % NBUF], gsems[n % NBUF])
    for c in range(max(0, NCH - NBUF), NCH):
        wrs[c].wait()


def _make_router():
    return pl.pallas_call(
        _router_body,
        grid=(S // TB,),
        in_specs=[
            pl.BlockSpec((TB, D), lambda i: (i, 0)),
            pl.BlockSpec((D, LW), lambda i: (0, 0)),
        ],
        out_specs=[
            pl.BlockSpec((TB, 1), lambda i: (i, 0)),
            pl.BlockSpec((1, 1), lambda i: (0, 0)),
            pl.BlockSpec((TB, DP), lambda i: (i, 0)),
        ],
        out_shape=[
            jax.ShapeDtypeStruct((S, 1), jnp.int32),
            jax.ShapeDtypeStruct((1, 1), jnp.float32),
            jax.ShapeDtypeStruct((S, DP), jnp.float32),
        ],
        scratch_shapes=[pltpu.VMEM((8, LW), jnp.float32)],
        compiler_params=pltpu.CompilerParams(
            dimension_semantics=("arbitrary",)),
    )


def _make_ffn():
    return pl.pallas_call(
        _ffn_body,
        grid=(E, NJ),
        in_specs=[
            pl.BlockSpec((1, RPE, DP), lambda e, j: (e, 0, 0)),
            pl.BlockSpec((1, DFB, D), lambda e, j: (e, j, 0)),
            pl.BlockSpec((1, 1, DFB), lambda e, j: (e, 0, j)),
            pl.BlockSpec((1, D, DFB), lambda e, j: (e, 0, j)),
            pl.BlockSpec((1, 1, D), lambda e, j: (e, 0, 0)),
        ],
        out_specs=pl.BlockSpec((1, RPE, DP), lambda e, j: (e, 0, 0)),
        out_shape=jax.ShapeDtypeStruct((E, RPE, DP), jnp.float32),
        scratch_shapes=[pltpu.VMEM((RPE, D), jnp.float32)],
        compiler_params=pltpu.CompilerParams(
            dimension_semantics=("arbitrary", "arbitrary")),
    )


def _sc_mesh():
    return plsc.VectorSubcoreMesh(core_axis_name="c", subcore_axis_name="s")


def _make_dispatch():
    return functools.partial(
        pl.kernel,
        out_type=jax.ShapeDtypeStruct((NROWS, DP), jnp.float32),
        mesh=_sc_mesh(),
        scratch_types=[
            pltpu.VMEM((NCH, CH), jnp.int32),
            pltpu.VMEM((CH, DP), jnp.float32),
            pltpu.VMEM((CH, DP), jnp.float32),
            pltpu.VMEM((CH, DP), jnp.float32),
            pltpu.SemaphoreType.DMA,
            pltpu.SemaphoreType.DMA,
            pltpu.SemaphoreType.DMA,
            pltpu.SemaphoreType.DMA,
            pltpu.SemaphoreType.DMA,
            pltpu.SemaphoreType.DMA,
        ],
    )(_dispatch_body)


def _make_combine():
    return functools.partial(
        pl.kernel,
        out_type=jax.ShapeDtypeStruct((S, DP), jnp.float32),
        mesh=_sc_mesh(),
        scratch_types=[
            pltpu.VMEM((NCH, CH), jnp.int32),
            pltpu.VMEM((CH, DP), jnp.float32),
            pltpu.VMEM((CH, DP), jnp.float32),
            pltpu.VMEM((CH, DP), jnp.float32),
            pltpu.SemaphoreType.DMA,
            pltpu.SemaphoreType.DMA,
            pltpu.SemaphoreType.DMA,
            pltpu.SemaphoreType.DMA,
            pltpu.SemaphoreType.DMA,
            pltpu.SemaphoreType.DMA,
        ],
    )(_combine_body)


def kernel(H, gate_W, fc1_w, fc1_b, fc2_w, fc2_b):
    b, t, d = H.shape
    h2 = H.reshape(S, D)
    gwt = jnp.zeros((D, LW), jnp.float32).at[:, :E].set(gate_W.T)

    dst, loss, hbf = _make_router()(h2, gwt)
    dst3 = dst.reshape(NW, NCH, CH)

    x = _make_dispatch()(hbf, dst3)
    y = _make_ffn()(x.reshape(E, RPE, DP), fc1_w,
                    fc1_b.reshape(E, 1, DFF), fc2_w, fc2_b.reshape(E, 1, D))
    out = _make_combine()(y.reshape(NROWS, DP), dst3)
    w = lax.bitcast_convert_type(out, jnp.uint32)
    lo = lax.bitcast_convert_type((w & 0xFFFF).astype(jnp.uint16), jnp.bfloat16)
    hi = lax.bitcast_convert_type((w >> 16).astype(jnp.uint16), jnp.bfloat16)
    outf = jnp.concatenate(
        [lo.astype(jnp.float32), hi.astype(jnp.float32)], axis=-1)
    return outf.reshape(b, t, d), loss[0, 0]


# speedup vs baseline: 1.3960x; 1.0348x over previous
import functools

import jax
import jax.numpy as jnp
from jax import lax
from jax.experimental import pallas as pl
from jax.experimental.pallas import tpu as pltpu
from jax.experimental.pallas import tpu_sc as plsc

S = 4096
D = 1024
E = 8
DFF = 4096
CAP = 640
RPE = 656
NROWS = E * RPE
SENT = 648
TB = 512
LW = 128
NJ = 4
DFB = DFF // NJ
DP = D // 2

NW = 32
CH = 32
NCH = (S // NW) // CH
NBUF = 3

ROUTER_WEIGHT = 0.001
EXPERTS_WEIGHT = 0.01


def _pack_pair(lo_bf, hi_bf):
    lo = pltpu.bitcast(lo_bf, jnp.uint16).astype(jnp.uint32)
    hi = pltpu.bitcast(hi_bf, jnp.uint16).astype(jnp.uint32)
    return pltpu.bitcast(lo | (hi << 16), jnp.float32)


def _unpack_pair(x_f32):
    w = pltpu.bitcast(x_f32, jnp.uint32)
    lo = pltpu.bitcast((w & 0xFFFF).astype(jnp.uint16), jnp.bfloat16)
    hi = pltpu.bitcast((w >> 16).astype(jnp.uint16), jnp.bfloat16)
    return lo, hi


def _router_body(h_ref, gw_ref, dst_ref, loss_ref, hbf_ref, acc_ref):
    i = pl.program_id(0)
    nb = pl.num_programs(0)

    @pl.when(i == 0)
    def _():
        acc_ref[...] = jnp.zeros_like(acc_ref)

    h = h_ref[...]
    hbf_ref[...] = h.astype(jnp.bfloat16)
    logits = jnp.dot(h, gw_ref[...], preferred_element_type=jnp.float32)
    lane = lax.broadcasted_iota(jnp.int32, (TB, LW), 1)
    neg_inf = jnp.float32(-jnp.inf)
    logits = jnp.where(lane < E, logits, neg_inf)

    m1 = jnp.max(logits, axis=1, keepdims=True)
    i1 = jnp.min(jnp.where(logits == m1, lane, LW), axis=1, keepdims=True)
    l2 = jnp.where(lane == i1, neg_inf, logits)
    m2 = jnp.max(l2, axis=1, keepdims=True)
    i2 = jnp.min(jnp.where(l2 == m2, lane, LW), axis=1, keepdims=True)

    ex = jnp.exp(logits - m1)
    se = jnp.sum(ex, axis=1, keepdims=True)
    probs = ex / se
    lse = m1 + jnp.log(se)

    oh1 = (lane == i1).astype(jnp.float32)
    oh2 = (lane == i2).astype(jnp.float32)
    d_oh = oh1 + oh2

    r_i = lax.broadcasted_iota(jnp.int32, (TB, TB), 0)
    c_i = lax.broadcasted_iota(jnp.int32, (TB, TB), 1)
    tri = (r_i >= c_i).astype(jnp.float32)

    pos = jnp.dot(tri, d_oh, preferred_element_type=jnp.float32) + acc_ref[0:1, :]
    pos1 = jnp.sum(pos * oh1, axis=1, keepdims=True)
    pos2 = jnp.sum(pos * oh2, axis=1, keepdims=True)
    cand1 = jnp.where(pos1 <= CAP, i1, -1)
    cand2 = jnp.where(pos2 <= CAP, i2, -1)
    assigned = jnp.maximum(cand1, cand2)

    a_oh = (lane == assigned).astype(jnp.float32)
    spos = jnp.dot(tri, a_oh, preferred_element_type=jnp.float32) + acc_ref[1:2, :]
    slot = jnp.sum(spos * a_oh, axis=1, keepdims=True).astype(jnp.int32) - 1
    dst_ref[...] = jnp.where(assigned >= 0, assigned * RPE + slot, SENT)

    acc_ref[0:1, :] = acc_ref[0:1, :] + jnp.sum(d_oh, axis=0, keepdims=True)
    acc_ref[1:2, :] = acc_ref[1:2, :] + jnp.sum(a_oh, axis=0, keepdims=True)
    acc_ref[2:3, :] = acc_ref[2:3, :] + jnp.sum(probs, axis=0, keepdims=True)
    acc_ref[3:4, :] = acc_ref[3:4, :] + jnp.sum(lse * lse)

    @pl.when(i == nb - 1)
    def _():
        sf = jnp.float32(S)
        load = acc_ref[0:1, :] / sf
        imp = acc_ref[2:3, :] / sf
        bal = EXPERTS_WEIGHT * jnp.sum(load * imp)
        rl = ROUTER_WEIGHT * jnp.max(acc_ref[3:4, :]) / sf
        loss_ref[...] = jnp.reshape(rl + bal, (1, 1))


def _ffn_body(dst_ref, hbf_ref, w1_ref, b1_ref, w2_ref, b2_ref, y_ref,
              xe_ref, acc_ref):
    e = pl.program_id(0)
    j = pl.program_id(1)

    @pl.when(j == 0)
    def _():
        rows = lax.broadcasted_iota(jnp.int32, (S, RPE), 1) + e * RPE
        oneh = (dst_ref[...] == rows).astype(jnp.bfloat16)
        xe_ref[...] = lax.dot_general(
            oneh, hbf_ref[...], (((0,), (0,)), ((), ())),
            preferred_element_type=jnp.float32).astype(jnp.bfloat16)

    x = xe_ref[...]
    h1 = lax.dot_general(x, w1_ref[0].astype(jnp.bfloat16),
                         (((1,), (1,)), ((), ())),
                         preferred_element_type=jnp.float32)
    h1 = h1 + b1_ref[0]
    h1 = 0.5 * h1 * (1.0 + lax.erf(h1 * jnp.float32(0.7071067811865476)))
    contrib = lax.dot_general(h1.astype(jnp.bfloat16),
                              w2_ref[0].astype(jnp.bfloat16),
                              (((1,), (1,)), ((), ())),
                              preferred_element_type=jnp.float32)

    @pl.when(j == 0)
    def _():
        acc_ref[...] = contrib + b2_ref[0]

    @pl.when(j > 0)
    def _():
        acc_ref[...] = acc_ref[...] + contrib

    @pl.when(j == NJ - 1)
    def _():
        row = lax.broadcasted_iota(jnp.int32, (RPE, D), 0)
        yv = jnp.where(row < CAP, acc_ref[...], 0.0).astype(jnp.bfloat16)
        y_ref[0] = _pack_pair(yv[:, :DP], yv[:, DP:])


def _sc_wid():
    return lax.axis_index("s") * 2 + lax.axis_index("c")


def _combine_body(y_hbm, dst_hbm, o_hbm, idx_v, b0, b1, b2, s0, s1, s2,
                  t0, t1, t2):
    wid = _sc_wid()
    base = wid * (NCH * CH)
    bufs, gsems, wsems = (b0, b1, b2), (s0, s1, s2), (t0, t1, t2)
    pltpu.sync_copy(dst_hbm.at[wid], idx_v)
    gts, wrs = [None] * NCH, [None] * NCH
    for c in range(NBUF):
        gts[c] = pltpu.async_copy(
            y_hbm.at[idx_v.at[c]], bufs[c], gsems[c])
    for c in range(NCH):
        gts[c].wait()
        wrs[c] = pltpu.async_copy(
            bufs[c % NBUF], o_hbm.at[pl.ds(base + c * CH, CH)],
            wsems[c % NBUF])
        n = c + NBUF
        if n < NCH:
            wrs[c].wait()
            gts[n] = pltpu.async_copy(
                y_hbm.at[idx_v.at[n]], bufs[n % NBUF], gsems[n % NBUF])
    for c in range(max(0, NCH - NBUF), NCH):
        wrs[c].wait()


def _make_router():
    return pl.pallas_call(
        _router_body,
        grid=(S // TB,),
        in_specs=[
            pl.BlockSpec((TB, D), lambda i: (i, 0)),
            pl.BlockSpec((D, LW), lambda i: (0, 0)),
        ],
        out_specs=[
            pl.BlockSpec((TB, 1), lambda i: (i, 0)),
            pl.BlockSpec((1, 1), lambda i: (0, 0)),
            pl.BlockSpec((TB, D), lambda i: (i, 0)),
        ],
        out_shape=[
            jax.ShapeDtypeStruct((S, 1), jnp.int32),
            jax.ShapeDtypeStruct((1, 1), jnp.float32),
            jax.ShapeDtypeStruct((S, D), jnp.bfloat16),
        ],
        scratch_shapes=[pltpu.VMEM((8, LW), jnp.float32)],
        compiler_params=pltpu.CompilerParams(
            dimension_semantics=("arbitrary",)),
    )


def _make_ffn():
    return pl.pallas_call(
        _ffn_body,
        grid=(E, NJ),
        in_specs=[
            pl.BlockSpec((S, 1), lambda e, j: (0, 0)),
            pl.BlockSpec((S, D), lambda e, j: (0, 0)),
            pl.BlockSpec((1, DFB, D), lambda e, j: (e, j, 0)),
            pl.BlockSpec((1, 1, DFB), lambda e, j: (e, 0, j)),
            pl.BlockSpec((1, D, DFB), lambda e, j: (e, 0, j)),
            pl.BlockSpec((1, 1, D), lambda e, j: (e, 0, 0)),
        ],
        out_specs=pl.BlockSpec((1, RPE, DP), lambda e, j: (e, 0, 0)),
        out_shape=jax.ShapeDtypeStruct((E, RPE, DP), jnp.float32),
        scratch_shapes=[pltpu.VMEM((RPE, D), jnp.bfloat16),
                        pltpu.VMEM((RPE, D), jnp.float32)],
        compiler_params=pltpu.CompilerParams(
            dimension_semantics=("arbitrary", "arbitrary")),
    )


def _sc_mesh():
    return plsc.VectorSubcoreMesh(core_axis_name="c", subcore_axis_name="s")


def _make_combine():
    return functools.partial(
        pl.kernel,
        out_type=jax.ShapeDtypeStruct((S, DP), jnp.float32),
        mesh=_sc_mesh(),
        scratch_types=[
            pltpu.VMEM((NCH, CH), jnp.int32),
            pltpu.VMEM((CH, DP), jnp.float32),
            pltpu.VMEM((CH, DP), jnp.float32),
            pltpu.VMEM((CH, DP), jnp.float32),
            pltpu.SemaphoreType.DMA,
            pltpu.SemaphoreType.DMA,
            pltpu.SemaphoreType.DMA,
            pltpu.SemaphoreType.DMA,
            pltpu.SemaphoreType.DMA,
            pltpu.SemaphoreType.DMA,
        ],
    )(_combine_body)


def kernel(H, gate_W, fc1_w, fc1_b, fc2_w, fc2_b):
    b, t, d = H.shape
    h2 = H.reshape(S, D)
    gwt = jnp.zeros((D, LW), jnp.float32).at[:, :E].set(gate_W.T)

    dst, loss, hbf = _make_router()(h2, gwt)
    dst3 = dst.reshape(NW, NCH, CH)

    y = _make_ffn()(dst, hbf, fc1_w,
                    fc1_b.reshape(E, 1, DFF), fc2_w, fc2_b.reshape(E, 1, D))
    out = _make_combine()(y.reshape(NROWS, DP), dst3)
    w = lax.bitcast_convert_type(out, jnp.uint32)
    lo = lax.bitcast_convert_type((w & 0xFFFF).astype(jnp.uint16), jnp.bfloat16)
    hi = lax.bitcast_convert_type((w >> 16).astype(jnp.uint16), jnp.bfloat16)
    outf = jnp.concatenate(
        [lo.astype(jnp.float32), hi.astype(jnp.float32)], axis=-1)
    return outf.reshape(b, t, d), loss[0, 0]


# speedup vs baseline: 1.4078x; 1.0084x over previous
import functools

import jax
import jax.numpy as jnp
from jax import lax
from jax.experimental import pallas as pl
from jax.experimental.pallas import tpu as pltpu
from jax.experimental.pallas import tpu_sc as plsc

S = 4096
D = 1024
E = 8
DFF = 4096
CAP = 640
RPE = 656
NROWS = E * RPE
SENT = 648
TB = 512
LW = 128
NJ = 4
DFB = DFF // NJ
DP = D // 2

NW = 32
CH = 64
NCH = (S // NW) // CH
NBUF = 2

ROUTER_WEIGHT = 0.001
EXPERTS_WEIGHT = 0.01


def _pack_pair(lo_bf, hi_bf):
    lo = pltpu.bitcast(lo_bf, jnp.uint16).astype(jnp.uint32)
    hi = pltpu.bitcast(hi_bf, jnp.uint16).astype(jnp.uint32)
    return pltpu.bitcast(lo | (hi << 16), jnp.float32)


def _unpack_pair(x_f32):
    w = pltpu.bitcast(x_f32, jnp.uint32)
    lo = pltpu.bitcast((w & 0xFFFF).astype(jnp.uint16), jnp.bfloat16)
    hi = pltpu.bitcast((w >> 16).astype(jnp.uint16), jnp.bfloat16)
    return lo, hi


def _router_body(h_ref, gw_ref, dst_ref, loss_ref, hbf_ref, acc_ref):
    i = pl.program_id(0)
    nb = pl.num_programs(0)

    @pl.when(i == 0)
    def _():
        acc_ref[...] = jnp.zeros_like(acc_ref)

    h = h_ref[...]
    hbf_ref[...] = h.astype(jnp.bfloat16)
    logits = jnp.dot(h, gw_ref[...], preferred_element_type=jnp.float32)
    lane = lax.broadcasted_iota(jnp.int32, (TB, LW), 1)
    neg_inf = jnp.float32(-jnp.inf)
    logits = jnp.where(lane < E, logits, neg_inf)

    m1 = jnp.max(logits, axis=1, keepdims=True)
    i1 = jnp.min(jnp.where(logits == m1, lane, LW), axis=1, keepdims=True)
    l2 = jnp.where(lane == i1, neg_inf, logits)
    m2 = jnp.max(l2, axis=1, keepdims=True)
    i2 = jnp.min(jnp.where(l2 == m2, lane, LW), axis=1, keepdims=True)

    ex = jnp.exp(logits - m1)
    se = jnp.sum(ex, axis=1, keepdims=True)
    probs = ex / se
    lse = m1 + jnp.log(se)

    oh1 = (lane == i1).astype(jnp.float32)
    oh2 = (lane == i2).astype(jnp.float32)
    d_oh = oh1 + oh2

    r_i = lax.broadcasted_iota(jnp.int32, (TB, TB), 0)
    c_i = lax.broadcasted_iota(jnp.int32, (TB, TB), 1)
    tri = (r_i >= c_i).astype(jnp.float32)

    pos = jnp.dot(tri, d_oh, preferred_element_type=jnp.float32) + acc_ref[0:1, :]
    pos1 = jnp.sum(pos * oh1, axis=1, keepdims=True)
    pos2 = jnp.sum(pos * oh2, axis=1, keepdims=True)
    cand1 = jnp.where(pos1 <= CAP, i1, -1)
    cand2 = jnp.where(pos2 <= CAP, i2, -1)
    assigned = jnp.maximum(cand1, cand2)

    a_oh = (lane == assigned).astype(jnp.float32)
    spos = jnp.dot(tri, a_oh, preferred_element_type=jnp.float32) + acc_ref[1:2, :]
    slot = jnp.sum(spos * a_oh, axis=1, keepdims=True).astype(jnp.int32) - 1
    dst_ref[...] = jnp.where(assigned >= 0, assigned * RPE + slot, SENT)

    acc_ref[0:1, :] = acc_ref[0:1, :] + jnp.sum(d_oh, axis=0, keepdims=True)
    acc_ref[1:2, :] = acc_ref[1:2, :] + jnp.sum(a_oh, axis=0, keepdims=True)
    acc_ref[2:3, :] = acc_ref[2:3, :] + jnp.sum(probs, axis=0, keepdims=True)
    acc_ref[3:4, :] = acc_ref[3:4, :] + jnp.sum(lse * lse)

    @pl.when(i == nb - 1)
    def _():
        sf = jnp.float32(S)
        load = acc_ref[0:1, :] / sf
        imp = acc_ref[2:3, :] / sf
        bal = EXPERTS_WEIGHT * jnp.sum(load * imp)
        rl = ROUTER_WEIGHT * jnp.max(acc_ref[3:4, :]) / sf
        loss_ref[...] = jnp.reshape(rl + bal, (1, 1))


def _ffn_body(dst_ref, hbf_ref, w1_ref, b1_ref, w2_ref, b2_ref, y_ref,
              xe_ref, acc_ref):
    e = pl.program_id(0)
    j = pl.program_id(1)

    @pl.when(j == 0)
    def _():
        rows = lax.broadcasted_iota(jnp.int32, (S, RPE), 1) + e * RPE
        oneh = (dst_ref[...] == rows).astype(jnp.bfloat16)
        xe_ref[...] = lax.dot_general(
            oneh, hbf_ref[...], (((0,), (0,)), ((), ())),
            preferred_element_type=jnp.float32).astype(jnp.bfloat16)

    x = xe_ref[...]
    h1 = lax.dot_general(x, w1_ref[0].astype(jnp.bfloat16),
                         (((1,), (1,)), ((), ())),
                         preferred_element_type=jnp.float32)
    h1 = h1 + b1_ref[0]
    h1 = 0.5 * h1 * (1.0 + lax.erf(h1 * jnp.float32(0.7071067811865476)))
    contrib = lax.dot_general(h1.astype(jnp.bfloat16),
                              w2_ref[0].astype(jnp.bfloat16),
                              (((1,), (1,)), ((), ())),
                              preferred_element_type=jnp.float32)

    @pl.when(j == 0)
    def _():
        acc_ref[...] = contrib + b2_ref[0]

    @pl.when(j > 0)
    def _():
        acc_ref[...] = acc_ref[...] + contrib

    @pl.when(j == NJ - 1)
    def _():
        row = lax.broadcasted_iota(jnp.int32, (RPE, D), 0)
        yv = jnp.where(row < CAP, acc_ref[...], 0.0).astype(jnp.bfloat16)
        y_ref[0] = _pack_pair(yv[:, :DP], yv[:, DP:])


def _sc_wid():
    return lax.axis_index("s") * 2 + lax.axis_index("c")


def _combine_body(y_hbm, dst_hbm, o_hbm, idx_v, b0, b1, b2, s0, s1, s2,
                  t0, t1, t2):
    wid = _sc_wid()
    base = wid * (NCH * CH)
    bufs, gsems, wsems = (b0, b1, b2), (s0, s1, s2), (t0, t1, t2)
    pltpu.sync_copy(dst_hbm.at[wid], idx_v)
    gts, wrs = [None] * NCH, [None] * NCH
    for c in range(min(NBUF, NCH)):
        gts[c] = pltpu.async_copy(
            y_hbm.at[idx_v.at[c]], bufs[c], gsems[c])
    for c in range(NCH):
        gts[c].wait()
        wrs[c] = pltpu.async_copy(
            bufs[c % NBUF], o_hbm.at[pl.ds(base + c * CH, CH)],
            wsems[c % NBUF])
        n = c + NBUF
        if n < NCH:
            wrs[c].wait()
            gts[n] = pltpu.async_copy(
                y_hbm.at[idx_v.at[n]], bufs[n % NBUF], gsems[n % NBUF])
    for c in range(max(0, NCH - NBUF), NCH):
        wrs[c].wait()


def _make_router():
    return pl.pallas_call(
        _router_body,
        grid=(S // TB,),
        in_specs=[
            pl.BlockSpec((TB, D), lambda i: (i, 0)),
            pl.BlockSpec((D, LW), lambda i: (0, 0)),
        ],
        out_specs=[
            pl.BlockSpec((TB, 1), lambda i: (i, 0)),
            pl.BlockSpec((1, 1), lambda i: (0, 0)),
            pl.BlockSpec((TB, D), lambda i: (i, 0)),
        ],
        out_shape=[
            jax.ShapeDtypeStruct((S, 1), jnp.int32),
            jax.ShapeDtypeStruct((1, 1), jnp.float32),
            jax.ShapeDtypeStruct((S, D), jnp.bfloat16),
        ],
        scratch_shapes=[pltpu.VMEM((8, LW), jnp.float32)],
        compiler_params=pltpu.CompilerParams(
            dimension_semantics=("arbitrary",)),
    )


def _make_ffn():
    return pl.pallas_call(
        _ffn_body,
        grid=(E, NJ),
        in_specs=[
            pl.BlockSpec((S, 1), lambda e, j: (0, 0)),
            pl.BlockSpec((S, D), lambda e, j: (0, 0)),
            pl.BlockSpec((1, DFB, D), lambda e, j: (e, j, 0)),
            pl.BlockSpec((1, 1, DFB), lambda e, j: (e, 0, j)),
            pl.BlockSpec((1, D, DFB), lambda e, j: (e, 0, j)),
            pl.BlockSpec((1, 1, D), lambda e, j: (e, 0, 0)),
        ],
        out_specs=pl.BlockSpec((1, RPE, DP), lambda e, j: (e, 0, 0)),
        out_shape=jax.ShapeDtypeStruct((E, RPE, DP), jnp.float32),
        scratch_shapes=[pltpu.VMEM((RPE, D), jnp.bfloat16),
                        pltpu.VMEM((RPE, D), jnp.float32)],
        compiler_params=pltpu.CompilerParams(
            dimension_semantics=("arbitrary", "arbitrary")),
    )


def _sc_mesh():
    return plsc.VectorSubcoreMesh(core_axis_name="c", subcore_axis_name="s")


def _make_combine():
    return functools.partial(
        pl.kernel,
        out_type=jax.ShapeDtypeStruct((S, DP), jnp.float32),
        mesh=_sc_mesh(),
        scratch_types=[
            pltpu.VMEM((NCH, CH), jnp.int32),
            pltpu.VMEM((CH, DP), jnp.float32),
            pltpu.VMEM((CH, DP), jnp.float32),
            pltpu.VMEM((CH, DP), jnp.float32),
            pltpu.SemaphoreType.DMA,
            pltpu.SemaphoreType.DMA,
            pltpu.SemaphoreType.DMA,
            pltpu.SemaphoreType.DMA,
            pltpu.SemaphoreType.DMA,
            pltpu.SemaphoreType.DMA,
        ],
    )(_combine_body)


def kernel(H, gate_W, fc1_w, fc1_b, fc2_w, fc2_b):
    b, t, d = H.shape
    h2 = H.reshape(S, D)
    gwt = jnp.zeros((D, LW), jnp.float32).at[:, :E].set(gate_W.T)

    dst, loss, hbf = _make_router()(h2, gwt)
    dst3 = dst.reshape(NW, NCH, CH)

    y = _make_ffn()(dst, hbf, fc1_w,
                    fc1_b.reshape(E, 1, DFF), fc2_w, fc2_b.reshape(E, 1, D))
    out = _make_combine()(y.reshape(NROWS, DP), dst3)
    w = lax.bitcast_convert_type(out, jnp.uint32)
    lo = lax.bitcast_convert_type((w & 0xFFFF).astype(jnp.uint16), jnp.bfloat16)
    hi = lax.bitcast_convert_type((w >> 16).astype(jnp.uint16), jnp.bfloat16)
    outf = jnp.concatenate(
        [lo.astype(jnp.float32), hi.astype(jnp.float32)], axis=-1)
    return outf.reshape(b, t, d), loss[0, 0]
